# Initial kernel scaffold; baseline (speedup 1.0000x reference)
import jax
import jax.numpy as jnp
from jax.experimental import pallas as pl


def kernel(x, edge_index, edge_weight, eps, W0, W_mean, W_std):
    raise NotImplementedError("write your pallas kernel here")


# speedup vs baseline: 5.5306x; 5.5306x over previous
import functools

import jax
import jax.numpy as jnp
from jax import lax
from jax.experimental import pallas as pl
from jax.experimental.pallas import tpu as pltpu
from jax.experimental.pallas import tpu_sc as plsc

N = 10000
E = 320000
D = 128
H1 = 32
H2 = 16

NC = 2
NS = 16
NW = NC * NS
CHUNK = 128
CHUNKS = 79
EPW = CHUNKS * CHUNK
E_PAD = NW * EPW
ROWS_PER_SUB = 632
N_PAD = NS * ROWS_PER_SUB


def _spmm_body(table_hbm, src_hbm, dst_hbm, w_hbm, zeros_hbm, out_hbm,
               src_v, dst_v, w_v, rows_v, accum_sh, sem):
    c = lax.axis_index("c")
    s = lax.axis_index("s")
    wid = s * NC + c

    pltpu.sync_copy(src_hbm.at[wid], src_v)
    pltpu.sync_copy(dst_hbm.at[wid], dst_v)
    pltpu.sync_copy(w_hbm.at[wid], w_v)

    pltpu.sync_copy(zeros_hbm,
                    accum_sh.at[pl.ds(s * ROWS_PER_SUB, ROWS_PER_SUB)])
    plsc.subcore_barrier()

    def chunk_body(j, carry):
        pltpu.async_copy(table_hbm.at[src_v.at[j]], rows_v, sem).wait()
        jj = jnp.full((16,), j * CHUNK, jnp.int32)
        for e in range(CHUNK):
            wv = plsc.load_gather(w_v, [jj + e])
            rows_v[e, pl.ds(0, 16)] = rows_v[e, pl.ds(0, 16)] * wv
            rows_v[e, pl.ds(16, 16)] = rows_v[e, pl.ds(16, 16)] * wv
        pltpu.sync_copy(rows_v, accum_sh.at[dst_v.at[j]], add=True)
        return carry

    lax.fori_loop(0, CHUNKS, chunk_body, 0)
    plsc.subcore_barrier()

    pltpu.sync_copy(accum_sh.at[pl.ds(s * ROWS_PER_SUB, ROWS_PER_SUB)],
                    out_hbm.at[c, pl.ds(s * ROWS_PER_SUB, ROWS_PER_SUB)])


def _spmm(table, srcp, dstp, wp, zeros):
    mesh = plsc.VectorSubcoreMesh(core_axis_name="c", subcore_axis_name="s")
    f = pl.kernel(
        _spmm_body,
        out_type=jax.ShapeDtypeStruct((NC, N_PAD, H1), jnp.float32),
        mesh=mesh,
        scratch_types=[
            pltpu.VMEM((CHUNKS, CHUNK), jnp.int32),
            pltpu.VMEM((CHUNKS, CHUNK), jnp.int32),
            pltpu.VMEM((EPW,), jnp.float32),
            pltpu.VMEM((CHUNK, H1), jnp.float32),
            pltpu.VMEM_SHARED((N_PAD, H1), jnp.float32),
            pltpu.SemaphoreType.DMA,
        ],
        compiler_params=pltpu.CompilerParams(
            needs_layout_passes=False, use_tc_tiling_on_sc=False),
    )
    return f(table, srcp, dstp, wp, zeros)


def _mm_body(x_ref, w_ref, o_ref):
    o_ref[...] = jax.lax.dot_general(
        x_ref[...], w_ref[...], (((1,), (0,)), ((), ())),
        preferred_element_type=jnp.float32,
        precision=jax.lax.Precision.HIGHEST)


def _relu_combine_body(p_ref, o_ref):
    o_ref[...] = jnp.maximum(p_ref[0, :N, :] + p_ref[1, :N, :], 0.0)


def _z_body(q_ref, wm_ref, ws_ref, eps_ref, z_ref):
    g = q_ref[0, :N, :] + q_ref[1, :N, :]
    zm = jax.lax.dot_general(g, wm_ref[...], (((1,), (0,)), ((), ())),
                             preferred_element_type=jnp.float32,
                             precision=jax.lax.Precision.HIGHEST)
    zl = jax.lax.dot_general(g, ws_ref[...], (((1,), (0,)), ((), ())),
                             preferred_element_type=jnp.float32,
                             precision=jax.lax.Precision.HIGHEST)
    z_ref[...] = zm + eps_ref[...] * jnp.exp(zl)


def _dec_body(a_ref, b_ref, o_ref):
    o_ref[...] = jax.lax.dot_general(
        a_ref[...], b_ref[...], (((1,), (1,)), ((), ())),
        preferred_element_type=jnp.float32)


BM = 1024
BN = 1024


@jax.jit
def kernel(x, edge_index, edge_weight, eps, W0, W_mean, W_std):
    src = edge_index[0].astype(jnp.int32)
    dst = edge_index[1].astype(jnp.int32)
    pad = E_PAD - E
    srcp = jnp.pad(src, (0, pad)).reshape(NW, CHUNKS, CHUNK)
    dstp = jnp.pad(dst, (0, pad)).reshape(NW, CHUNKS, CHUNK)
    wp = jnp.pad(edge_weight.astype(jnp.float32), (0, pad)).reshape(NW, EPW)
    zeros = jnp.zeros((ROWS_PER_SUB, H1), jnp.float32)

    h0 = pl.pallas_call(
        _mm_body,
        out_shape=jax.ShapeDtypeStruct((N, H1), jnp.float32),
    )(x, W0)

    p = _spmm(h0, srcp, dstp, wp, zeros)

    hidden1 = pl.pallas_call(
        _relu_combine_body,
        out_shape=jax.ShapeDtypeStruct((N, H1), jnp.float32),
    )(p)

    q = _spmm(hidden1, srcp, dstp, wp, zeros)

    z = pl.pallas_call(
        _z_body,
        out_shape=jax.ShapeDtypeStruct((N, H2), jnp.float32),
    )(q, W_mean, W_std, eps)

    rec = pl.pallas_call(
        _dec_body,
        grid=(pl.cdiv(N, BM), pl.cdiv(N, BN)),
        in_specs=[
            pl.BlockSpec((BM, H2), lambda i, j: (i, 0)),
            pl.BlockSpec((BN, H2), lambda i, j: (j, 0)),
        ],
        out_specs=pl.BlockSpec((BM, BN), lambda i, j: (i, j)),
        out_shape=jax.ShapeDtypeStruct((N, N), jnp.float32),
    )(z, z)

    return rec.reshape(-1)


# speedup vs baseline: 5.9841x; 1.0820x over previous
import functools

import jax
import jax.numpy as jnp
from jax import lax
from jax.experimental import pallas as pl
from jax.experimental.pallas import tpu as pltpu
from jax.experimental.pallas import tpu_sc as plsc

N = 10000
E = 320000
D = 128
H1 = 32
H2 = 16

NC = 2
NS = 16
NW = NC * NS
CHUNK = 128
CHUNKS = 80
EPW = CHUNKS * CHUNK
E_PAD = NW * EPW
ROWS_PER_SUB = 632
N_PAD = NS * ROWS_PER_SUB


def _spmm_body(table_hbm, src_hbm, dst_hbm, w_hbm, zeros_hbm, out_hbm,
               src_v, dst_v, w_v, rows_v0, rows_v1, accum_sh, sem0, sem1):
    c = lax.axis_index("c")
    s = lax.axis_index("s")
    wid = s * NC + c

    pltpu.sync_copy(src_hbm.at[wid], src_v)
    pltpu.sync_copy(dst_hbm.at[wid], dst_v)
    pltpu.sync_copy(w_hbm.at[wid], w_v)

    pltpu.sync_copy(zeros_hbm,
                    accum_sh.at[pl.ds(s * ROWS_PER_SUB, ROWS_PER_SUB)])
    plsc.subcore_barrier()

    def scale_and_scatter(j, rows_v):
        jj = jnp.full((16,), j * CHUNK, jnp.int32)
        for e in range(CHUNK):
            wv = plsc.load_gather(w_v, [jj + e])
            rows_v[e, pl.ds(0, 16)] = rows_v[e, pl.ds(0, 16)] * wv
            rows_v[e, pl.ds(16, 16)] = rows_v[e, pl.ds(16, 16)] * wv
        pltpu.sync_copy(rows_v, accum_sh.at[dst_v.at[j]], add=True)

    def gather(j, rows_v, sem):
        return pltpu.async_copy(table_hbm.at[src_v.at[j]], rows_v, sem)

    gather(0, rows_v0, sem0)

    def pair_body(h, carry):
        j0 = h * 2
        gather(j0 + 1, rows_v1, sem1)
        pltpu.make_async_copy(table_hbm.at[src_v.at[j0]], rows_v0, sem0).wait()
        scale_and_scatter(j0, rows_v0)

        @pl.when(h < CHUNKS // 2 - 1)
        def _():
            gather(j0 + 2, rows_v0, sem0)

        pltpu.make_async_copy(table_hbm.at[src_v.at[j0 + 1]],
                              rows_v1, sem1).wait()
        scale_and_scatter(j0 + 1, rows_v1)
        return carry

    lax.fori_loop(0, CHUNKS // 2, pair_body, 0)
    plsc.subcore_barrier()

    pltpu.sync_copy(accum_sh.at[pl.ds(s * ROWS_PER_SUB, ROWS_PER_SUB)],
                    out_hbm.at[c, pl.ds(s * ROWS_PER_SUB, ROWS_PER_SUB)])


def _spmm(table, srcp, dstp, wp, zeros):
    mesh = plsc.VectorSubcoreMesh(core_axis_name="c", subcore_axis_name="s")
    f = pl.kernel(
        _spmm_body,
        out_type=jax.ShapeDtypeStruct((NC, N_PAD, H1), jnp.float32),
        mesh=mesh,
        scratch_types=[
            pltpu.VMEM((CHUNKS, CHUNK), jnp.int32),
            pltpu.VMEM((CHUNKS, CHUNK), jnp.int32),
            pltpu.VMEM((EPW,), jnp.float32),
            pltpu.VMEM((CHUNK, H1), jnp.float32),
            pltpu.VMEM((CHUNK, H1), jnp.float32),
            pltpu.VMEM_SHARED((N_PAD, H1), jnp.float32),
            pltpu.SemaphoreType.DMA,
            pltpu.SemaphoreType.DMA,
        ],
        compiler_params=pltpu.CompilerParams(
            needs_layout_passes=False, use_tc_tiling_on_sc=False),
    )
    return f(table, srcp, dstp, wp, zeros)


def _mm_body(x_ref, w_ref, o_ref):
    o_ref[...] = jax.lax.dot_general(
        x_ref[...], w_ref[...], (((1,), (0,)), ((), ())),
        preferred_element_type=jnp.float32,
        precision=jax.lax.Precision.HIGHEST)


def _relu_combine_body(p_ref, o_ref):
    o_ref[...] = jnp.maximum(p_ref[0, :N, :] + p_ref[1, :N, :], 0.0)


def _z_body(q_ref, wm_ref, ws_ref, eps_ref, z_ref):
    g = q_ref[0, :N, :] + q_ref[1, :N, :]
    zm = jax.lax.dot_general(g, wm_ref[...], (((1,), (0,)), ((), ())),
                             preferred_element_type=jnp.float32,
                             precision=jax.lax.Precision.HIGHEST)
    zl = jax.lax.dot_general(g, ws_ref[...], (((1,), (0,)), ((), ())),
                             preferred_element_type=jnp.float32,
                             precision=jax.lax.Precision.HIGHEST)
    z_ref[...] = zm + eps_ref[...] * jnp.exp(zl)


def _dec_body(a_ref, b_ref, o_ref):
    o_ref[...] = jax.lax.dot_general(
        a_ref[...], b_ref[...], (((1,), (1,)), ((), ())),
        preferred_element_type=jnp.float32)


BM = 1024
BN = 1024


@jax.jit
def kernel(x, edge_index, edge_weight, eps, W0, W_mean, W_std):
    src = edge_index[0].astype(jnp.int32)
    dst = edge_index[1].astype(jnp.int32)
    pad = E_PAD - E
    srcp = jnp.pad(src, (0, pad)).reshape(NW, CHUNKS, CHUNK)
    dstp = jnp.pad(dst, (0, pad)).reshape(NW, CHUNKS, CHUNK)
    wp = jnp.pad(edge_weight.astype(jnp.float32), (0, pad)).reshape(NW, EPW)
    zeros = jnp.zeros((ROWS_PER_SUB, H1), jnp.float32)

    h0 = pl.pallas_call(
        _mm_body,
        out_shape=jax.ShapeDtypeStruct((N, H1), jnp.float32),
    )(x, W0)

    p = _spmm(h0, srcp, dstp, wp, zeros)

    hidden1 = pl.pallas_call(
        _relu_combine_body,
        out_shape=jax.ShapeDtypeStruct((N, H1), jnp.float32),
    )(p)

    q = _spmm(hidden1, srcp, dstp, wp, zeros)

    z = pl.pallas_call(
        _z_body,
        out_shape=jax.ShapeDtypeStruct((N, H2), jnp.float32),
    )(q, W_mean, W_std, eps)

    rec = pl.pallas_call(
        _dec_body,
        grid=(pl.cdiv(N, BM), pl.cdiv(N, BN)),
        in_specs=[
            pl.BlockSpec((BM, H2), lambda i, j: (i, 0)),
            pl.BlockSpec((BN, H2), lambda i, j: (j, 0)),
        ],
        out_specs=pl.BlockSpec((BM, BN), lambda i, j: (i, j)),
        out_shape=jax.ShapeDtypeStruct((N, N), jnp.float32),
    )(z, z)

    return rec.reshape(-1)


# speedup vs baseline: 6.1885x; 1.0341x over previous
import functools

import jax
import jax.numpy as jnp
from jax import lax
from jax.experimental import pallas as pl
from jax.experimental.pallas import tpu as pltpu
from jax.experimental.pallas import tpu_sc as plsc

N = 10000
E = 320000
D = 128
H1 = 32
H2 = 16

NC = 2
NS = 16
NW = NC * NS
CHUNK = 128
CHUNKS = 80
EPW = CHUNKS * CHUNK
E_PAD = NW * EPW
ROWS_PER_SUB = 632
N_PAD = NS * ROWS_PER_SUB


def _spmm_body(table_hbm, src_hbm, dst_hbm, w_hbm, zeros_hbm, out_hbm,
               src_v, dst_v, w_v, gbuf0, gbuf1, sbuf0, sbuf1, accum_sh,
               sem_g0, sem_g1, sem_s0, sem_s1):
    c = lax.axis_index("c")
    s = lax.axis_index("s")
    wid = s * NC + c

    pltpu.sync_copy(src_hbm.at[wid], src_v)
    pltpu.sync_copy(dst_hbm.at[wid], dst_v)
    pltpu.sync_copy(w_hbm.at[wid], w_v)

    pltpu.sync_copy(zeros_hbm,
                    accum_sh.at[pl.ds(s * ROWS_PER_SUB, ROWS_PER_SUB)])
    plsc.subcore_barrier()

    def scale(j, gbuf, sbuf):
        jj = jnp.full((16,), j * CHUNK, jnp.int32)

        @plsc.parallel_loop(0, CHUNK, 1, unroll=8)
        def _(e):
            wv = plsc.load_gather(w_v, [jj + e])
            sbuf[e, pl.ds(0, 16)] = gbuf[e, pl.ds(0, 16)] * wv
            sbuf[e, pl.ds(16, 16)] = gbuf[e, pl.ds(16, 16)] * wv

    def gather_start(j, gbuf, sem):
        pltpu.async_copy(table_hbm.at[src_v.at[j]], gbuf, sem)

    def gather_wait(j, gbuf, sem):
        pltpu.make_async_copy(table_hbm.at[src_v.at[j]], gbuf, sem).wait()

    def scatter_start(j, sbuf, sem):
        pltpu.async_copy(sbuf, accum_sh.at[dst_v.at[j]], sem, add=True)

    def scatter_wait(j, sbuf, sem):
        pltpu.make_async_copy(sbuf, accum_sh.at[dst_v.at[j]], sem).wait()

    HALF = CHUNKS // 2

    gather_start(0, gbuf0, sem_g0)
    gather_start(1, gbuf1, sem_g1)

    def pair_body(h, carry):
        j0 = h * 2
        j1 = j0 + 1

        gather_wait(j0, gbuf0, sem_g0)

        @pl.when(h > 0)
        def _():
            scatter_wait(j0 - 2, sbuf0, sem_s0)

        scale(j0, gbuf0, sbuf0)

        @pl.when(h < HALF - 1)
        def _():
            gather_start(j0 + 2, gbuf0, sem_g0)

        scatter_start(j0, sbuf0, sem_s0)

        gather_wait(j1, gbuf1, sem_g1)

        @pl.when(h > 0)
        def _():
            scatter_wait(j1 - 2, sbuf1, sem_s1)

        scale(j1, gbuf1, sbuf1)

        @pl.when(h < HALF - 1)
        def _():
            gather_start(j1 + 2, gbuf1, sem_g1)

        scatter_start(j1, sbuf1, sem_s1)
        return carry

    lax.fori_loop(0, HALF, pair_body, 0)
    scatter_wait(CHUNKS - 2, sbuf0, sem_s0)
    scatter_wait(CHUNKS - 1, sbuf1, sem_s1)
    plsc.subcore_barrier()

    pltpu.sync_copy(accum_sh.at[pl.ds(s * ROWS_PER_SUB, ROWS_PER_SUB)],
                    out_hbm.at[c, pl.ds(s * ROWS_PER_SUB, ROWS_PER_SUB)])


def _spmm(table, srcp, dstp, wp, zeros):
    mesh = plsc.VectorSubcoreMesh(core_axis_name="c", subcore_axis_name="s")
    f = pl.kernel(
        _spmm_body,
        out_type=jax.ShapeDtypeStruct((NC, N_PAD, H1), jnp.float32),
        mesh=mesh,
        scratch_types=[
            pltpu.VMEM((CHUNKS, CHUNK), jnp.int32),
            pltpu.VMEM((CHUNKS, CHUNK), jnp.int32),
            pltpu.VMEM((EPW,), jnp.float32),
            pltpu.VMEM((CHUNK, H1), jnp.float32),
            pltpu.VMEM((CHUNK, H1), jnp.float32),
            pltpu.VMEM((CHUNK, H1), jnp.float32),
            pltpu.VMEM((CHUNK, H1), jnp.float32),
            pltpu.VMEM_SHARED((N_PAD, H1), jnp.float32),
            pltpu.SemaphoreType.DMA,
            pltpu.SemaphoreType.DMA,
            pltpu.SemaphoreType.DMA,
            pltpu.SemaphoreType.DMA,
        ],
        compiler_params=pltpu.CompilerParams(
            needs_layout_passes=False, use_tc_tiling_on_sc=False),
    )
    return f(table, srcp, dstp, wp, zeros)


def _mm_body(x_ref, w_ref, o_ref):
    o_ref[...] = jax.lax.dot_general(
        x_ref[...], w_ref[...], (((1,), (0,)), ((), ())),
        preferred_element_type=jnp.float32,
        precision=jax.lax.Precision.HIGHEST)


def _relu_combine_body(p_ref, o_ref):
    o_ref[...] = jnp.maximum(p_ref[0, :N, :] + p_ref[1, :N, :], 0.0)


def _z_body(q_ref, wm_ref, ws_ref, eps_ref, z_ref):
    g = q_ref[0, :N, :] + q_ref[1, :N, :]
    zm = jax.lax.dot_general(g, wm_ref[...], (((1,), (0,)), ((), ())),
                             preferred_element_type=jnp.float32,
                             precision=jax.lax.Precision.HIGHEST)
    zl = jax.lax.dot_general(g, ws_ref[...], (((1,), (0,)), ((), ())),
                             preferred_element_type=jnp.float32,
                             precision=jax.lax.Precision.HIGHEST)
    z_ref[...] = zm + eps_ref[...] * jnp.exp(zl)


def _dec_body(a_ref, b_ref, o_ref):
    o_ref[...] = jax.lax.dot_general(
        a_ref[...], b_ref[...], (((1,), (1,)), ((), ())),
        preferred_element_type=jnp.float32)


BM = 1024
BN = 1024


@jax.jit
def kernel(x, edge_index, edge_weight, eps, W0, W_mean, W_std):
    src = edge_index[0].astype(jnp.int32)
    dst = edge_index[1].astype(jnp.int32)
    pad = E_PAD - E
    srcp = jnp.pad(src, (0, pad)).reshape(NW, CHUNKS, CHUNK)
    dstp = jnp.pad(dst, (0, pad)).reshape(NW, CHUNKS, CHUNK)
    wp = jnp.pad(edge_weight.astype(jnp.float32), (0, pad)).reshape(NW, EPW)
    zeros = jnp.zeros((ROWS_PER_SUB, H1), jnp.float32)

    h0 = pl.pallas_call(
        _mm_body,
        out_shape=jax.ShapeDtypeStruct((N, H1), jnp.float32),
    )(x, W0)

    p = _spmm(h0, srcp, dstp, wp, zeros)

    hidden1 = pl.pallas_call(
        _relu_combine_body,
        out_shape=jax.ShapeDtypeStruct((N, H1), jnp.float32),
    )(p)

    q = _spmm(hidden1, srcp, dstp, wp, zeros)

    z = pl.pallas_call(
        _z_body,
        out_shape=jax.ShapeDtypeStruct((N, H2), jnp.float32),
    )(q, W_mean, W_std, eps)

    rec = pl.pallas_call(
        _dec_body,
        grid=(pl.cdiv(N, BM), pl.cdiv(N, BN)),
        in_specs=[
            pl.BlockSpec((BM, H2), lambda i, j: (i, 0)),
            pl.BlockSpec((BN, H2), lambda i, j: (j, 0)),
        ],
        out_specs=pl.BlockSpec((BM, BN), lambda i, j: (i, j)),
        out_shape=jax.ShapeDtypeStruct((N, N), jnp.float32),
    )(z, z)

    return rec.reshape(-1)


# speedup vs baseline: 9.3130x; 1.5049x over previous
import functools

import jax
import jax.numpy as jnp
from jax import lax
from jax.experimental import pallas as pl
from jax.experimental.pallas import tpu as pltpu
from jax.experimental.pallas import tpu_sc as plsc

N = 10000
E = 320000
D = 128
H1 = 32
H2 = 16

NC = 2
NS = 16
NW = NC * NS
CHUNK = 128
CHUNKS = 80
EPW = CHUNKS * CHUNK
E_PAD = NW * EPW
ROWS_PER_SUB = 632
N_PAD = NS * ROWS_PER_SUB


def _spmm_body(table_hbm, src_hbm, dst_hbm, w_hbm, zeros_hbm, out_hbm,
               src_v, dst_v, w_v, gbuf0, gbuf1, sbuf0, sbuf1, accum_sh,
               sem_g0, sem_g1, sem_s0, sem_s1):
    c = lax.axis_index("c")
    s = lax.axis_index("s")
    wid = s * NC + c

    pltpu.sync_copy(src_hbm.at[wid], src_v)
    pltpu.sync_copy(dst_hbm.at[wid], dst_v)
    pltpu.sync_copy(w_hbm.at[wid], w_v)

    pltpu.sync_copy(zeros_hbm,
                    accum_sh.at[pl.ds(s * ROWS_PER_SUB, ROWS_PER_SUB)])
    plsc.subcore_barrier()

    def scale(j, gbuf, sbuf):
        jj = jnp.full((16,), j * CHUNK, jnp.int32)

        @plsc.parallel_loop(0, CHUNK, 1, unroll=8)
        def _(e):
            wv = plsc.load_gather(w_v, [jj + e])
            sbuf[e, pl.ds(0, 16)] = gbuf[e, pl.ds(0, 16)] * wv
            sbuf[e, pl.ds(16, 16)] = gbuf[e, pl.ds(16, 16)] * wv

    def gather_start(j, gbuf, sem):
        pltpu.async_copy(table_hbm.at[src_v.at[j]], gbuf, sem)

    def gather_wait(j, gbuf, sem):
        pltpu.make_async_copy(table_hbm.at[src_v.at[j]], gbuf, sem).wait()

    def scatter_start(j, sbuf, sem):
        pltpu.async_copy(sbuf, accum_sh.at[dst_v.at[j]], sem, add=True)

    def scatter_wait(j, sbuf, sem):
        pltpu.make_async_copy(sbuf, accum_sh.at[dst_v.at[j]], sem).wait()

    HALF = CHUNKS // 2

    gather_start(0, gbuf0, sem_g0)
    gather_start(1, gbuf1, sem_g1)

    def pair_body(h, carry):
        j0 = h * 2
        j1 = j0 + 1

        gather_wait(j0, gbuf0, sem_g0)

        @pl.when(h > 0)
        def _():
            scatter_wait(j0 - 2, sbuf0, sem_s0)

        scale(j0, gbuf0, sbuf0)

        @pl.when(h < HALF - 1)
        def _():
            gather_start(j0 + 2, gbuf0, sem_g0)

        scatter_start(j0, sbuf0, sem_s0)

        gather_wait(j1, gbuf1, sem_g1)

        @pl.when(h > 0)
        def _():
            scatter_wait(j1 - 2, sbuf1, sem_s1)

        scale(j1, gbuf1, sbuf1)

        @pl.when(h < HALF - 1)
        def _():
            gather_start(j1 + 2, gbuf1, sem_g1)

        scatter_start(j1, sbuf1, sem_s1)
        return carry

    lax.fori_loop(0, HALF, pair_body, 0)
    scatter_wait(CHUNKS - 2, sbuf0, sem_s0)
    scatter_wait(CHUNKS - 1, sbuf1, sem_s1)
    plsc.subcore_barrier()

    pltpu.sync_copy(accum_sh.at[pl.ds(s * ROWS_PER_SUB, ROWS_PER_SUB)],
                    out_hbm.at[c, pl.ds(s * ROWS_PER_SUB, ROWS_PER_SUB)])


def _spmm(table, srcp, dstp, wp, zeros):
    mesh = plsc.VectorSubcoreMesh(core_axis_name="c", subcore_axis_name="s")
    f = pl.kernel(
        _spmm_body,
        out_type=jax.ShapeDtypeStruct((NC, N_PAD, H1), jnp.float32),
        mesh=mesh,
        scratch_types=[
            pltpu.VMEM((CHUNKS, CHUNK), jnp.int32),
            pltpu.VMEM((CHUNKS, CHUNK), jnp.int32),
            pltpu.VMEM((EPW,), jnp.float32),
            pltpu.VMEM((CHUNK, H1), jnp.float32),
            pltpu.VMEM((CHUNK, H1), jnp.float32),
            pltpu.VMEM((CHUNK, H1), jnp.float32),
            pltpu.VMEM((CHUNK, H1), jnp.float32),
            pltpu.VMEM_SHARED((N_PAD, H1), jnp.float32),
            pltpu.SemaphoreType.DMA,
            pltpu.SemaphoreType.DMA,
            pltpu.SemaphoreType.DMA,
            pltpu.SemaphoreType.DMA,
        ],
        compiler_params=pltpu.CompilerParams(
            needs_layout_passes=False, use_tc_tiling_on_sc=False),
    )
    return f(table, srcp, dstp, wp, zeros)


def _mm_body(x_ref, w_ref, o_ref):
    o_ref[...] = jax.lax.dot_general(
        x_ref[...], w_ref[...], (((1,), (0,)), ((), ())),
        preferred_element_type=jnp.float32,
        precision=jax.lax.Precision.HIGHEST)


def _relu_combine_body(p_ref, o_ref):
    o_ref[...] = jnp.maximum(p_ref[0, :N, :] + p_ref[1, :N, :], 0.0)


def _z_body(q_ref, wm_ref, ws_ref, eps_ref, z_ref):
    g = q_ref[0, :N, :] + q_ref[1, :N, :]
    zm = jax.lax.dot_general(g, wm_ref[...], (((1,), (0,)), ((), ())),
                             preferred_element_type=jnp.float32,
                             precision=jax.lax.Precision.HIGHEST)
    zl = jax.lax.dot_general(g, ws_ref[...], (((1,), (0,)), ((), ())),
                             preferred_element_type=jnp.float32,
                             precision=jax.lax.Precision.HIGHEST)
    z_ref[...] = zm + eps_ref[...] * jnp.exp(zl)


def _dec_body(a_ref, b_ref, o_ref):
    m = jax.lax.dot_general(
        a_ref[...], b_ref[...], (((1,), (1,)), ((), ())),
        preferred_element_type=jnp.float32)
    for r in range(BM):
        o_ref[pl.ds(r * N, N)] = m[r, :]


BM = 64


@jax.jit
def kernel(x, edge_index, edge_weight, eps, W0, W_mean, W_std):
    src = edge_index[0].astype(jnp.int32)
    dst = edge_index[1].astype(jnp.int32)
    pad = E_PAD - E
    srcp = jnp.pad(src, (0, pad)).reshape(NW, CHUNKS, CHUNK)
    dstp = jnp.pad(dst, (0, pad)).reshape(NW, CHUNKS, CHUNK)
    wp = jnp.pad(edge_weight.astype(jnp.float32), (0, pad)).reshape(NW, EPW)
    zeros = jnp.zeros((ROWS_PER_SUB, H1), jnp.float32)

    h0 = pl.pallas_call(
        _mm_body,
        out_shape=jax.ShapeDtypeStruct((N, H1), jnp.float32),
    )(x, W0)

    p = _spmm(h0, srcp, dstp, wp, zeros)

    hidden1 = pl.pallas_call(
        _relu_combine_body,
        out_shape=jax.ShapeDtypeStruct((N, H1), jnp.float32),
    )(p)

    q = _spmm(hidden1, srcp, dstp, wp, zeros)

    z = pl.pallas_call(
        _z_body,
        out_shape=jax.ShapeDtypeStruct((N, H2), jnp.float32),
    )(q, W_mean, W_std, eps)

    rec = pl.pallas_call(
        _dec_body,
        grid=(pl.cdiv(N, BM),),
        in_specs=[
            pl.BlockSpec((BM, H2), lambda i: (i, 0)),
            pl.BlockSpec((N, H2), lambda i: (0, 0)),
        ],
        out_specs=pl.BlockSpec((BM * N,), lambda i: (i,)),
        out_shape=jax.ShapeDtypeStruct((N * N,), jnp.float32),
    )(z, z)

    return rec


# speedup vs baseline: 14.5982x; 1.5675x over previous
import functools

import jax
import jax.numpy as jnp
from jax import lax
from jax.experimental import pallas as pl
from jax.experimental.pallas import tpu as pltpu
from jax.experimental.pallas import tpu_sc as plsc

N = 10000
E = 320000
D = 128
H1 = 32
H2 = 16

NC = 2
NS = 16
NW = NC * NS
CHUNK = 128
CHUNKS = 80
EPW = CHUNKS * CHUNK
E_PAD = NW * EPW
ROWS_PER_SUB = 632
N_PAD = NS * ROWS_PER_SUB


def _spmm_body(table_hbm, src_hbm, dst_hbm, w_hbm, zeros_hbm, out_hbm,
               src_v, dst_v, w_v, gbuf0, gbuf1, sbuf0, sbuf1, accum_sh,
               sem_g0, sem_g1, sem_s0, sem_s1):
    c = lax.axis_index("c")
    s = lax.axis_index("s")
    wid = s * NC + c

    pltpu.sync_copy(src_hbm.at[wid], src_v)
    pltpu.sync_copy(dst_hbm.at[wid], dst_v)
    pltpu.sync_copy(w_hbm.at[wid], w_v)

    pltpu.sync_copy(zeros_hbm,
                    accum_sh.at[pl.ds(s * ROWS_PER_SUB, ROWS_PER_SUB)])
    plsc.subcore_barrier()

    def scale(j, gbuf, sbuf):
        jj = jnp.full((16,), j * CHUNK, jnp.int32)

        @plsc.parallel_loop(0, CHUNK, 1, unroll=8)
        def _(e):
            wv = plsc.load_gather(w_v, [jj + e])
            sbuf[e, pl.ds(0, 16)] = gbuf[e, pl.ds(0, 16)] * wv
            sbuf[e, pl.ds(16, 16)] = gbuf[e, pl.ds(16, 16)] * wv

    def gather_start(j, gbuf, sem):
        pltpu.async_copy(table_hbm.at[src_v.at[j]], gbuf, sem)

    def gather_wait(j, gbuf, sem):
        pltpu.make_async_copy(table_hbm.at[src_v.at[j]], gbuf, sem).wait()

    def scatter_start(j, sbuf, sem):
        pltpu.async_copy(sbuf, accum_sh.at[dst_v.at[j]], sem, add=True)

    def scatter_wait(j, sbuf, sem):
        pltpu.make_async_copy(sbuf, accum_sh.at[dst_v.at[j]], sem).wait()

    HALF = CHUNKS // 2

    gather_start(0, gbuf0, sem_g0)
    gather_start(1, gbuf1, sem_g1)

    def pair_body(h, carry):
        j0 = h * 2
        j1 = j0 + 1

        gather_wait(j0, gbuf0, sem_g0)

        @pl.when(h > 0)
        def _():
            scatter_wait(j0 - 2, sbuf0, sem_s0)

        scale(j0, gbuf0, sbuf0)

        @pl.when(h < HALF - 1)
        def _():
            gather_start(j0 + 2, gbuf0, sem_g0)

        scatter_start(j0, sbuf0, sem_s0)

        gather_wait(j1, gbuf1, sem_g1)

        @pl.when(h > 0)
        def _():
            scatter_wait(j1 - 2, sbuf1, sem_s1)

        scale(j1, gbuf1, sbuf1)

        @pl.when(h < HALF - 1)
        def _():
            gather_start(j1 + 2, gbuf1, sem_g1)

        scatter_start(j1, sbuf1, sem_s1)
        return carry

    lax.fori_loop(0, HALF, pair_body, 0)
    scatter_wait(CHUNKS - 2, sbuf0, sem_s0)
    scatter_wait(CHUNKS - 1, sbuf1, sem_s1)
    plsc.subcore_barrier()

    pltpu.sync_copy(accum_sh.at[pl.ds(s * ROWS_PER_SUB, ROWS_PER_SUB)],
                    out_hbm.at[c, pl.ds(s * ROWS_PER_SUB, ROWS_PER_SUB)])


def _spmm(table, srcp, dstp, wp, zeros):
    mesh = plsc.VectorSubcoreMesh(core_axis_name="c", subcore_axis_name="s")
    f = pl.kernel(
        _spmm_body,
        out_type=jax.ShapeDtypeStruct((NC, N_PAD, H1), jnp.float32),
        mesh=mesh,
        scratch_types=[
            pltpu.VMEM((CHUNKS, CHUNK), jnp.int32),
            pltpu.VMEM((CHUNKS, CHUNK), jnp.int32),
            pltpu.VMEM((EPW,), jnp.float32),
            pltpu.VMEM((CHUNK, H1), jnp.float32),
            pltpu.VMEM((CHUNK, H1), jnp.float32),
            pltpu.VMEM((CHUNK, H1), jnp.float32),
            pltpu.VMEM((CHUNK, H1), jnp.float32),
            pltpu.VMEM_SHARED((N_PAD, H1), jnp.float32),
            pltpu.SemaphoreType.DMA,
            pltpu.SemaphoreType.DMA,
            pltpu.SemaphoreType.DMA,
            pltpu.SemaphoreType.DMA,
        ],
        compiler_params=pltpu.CompilerParams(
            needs_layout_passes=False, use_tc_tiling_on_sc=False),
    )
    return f(table, srcp, dstp, wp, zeros)


def _mm_body(x_ref, w_ref, o_ref):
    o_ref[...] = jax.lax.dot_general(
        x_ref[...], w_ref[...], (((1,), (0,)), ((), ())),
        preferred_element_type=jnp.float32,
        precision=jax.lax.Precision.HIGHEST)


def _relu_combine_body(p_ref, o_ref):
    o_ref[...] = jnp.maximum(p_ref[0, :N, :] + p_ref[1, :N, :], 0.0)


def _z_body(q_ref, wm_ref, ws_ref, eps_ref, z_ref):
    g = q_ref[0, :N, :] + q_ref[1, :N, :]
    zm = jax.lax.dot_general(g, wm_ref[...], (((1,), (0,)), ((), ())),
                             preferred_element_type=jnp.float32,
                             precision=jax.lax.Precision.HIGHEST)
    zl = jax.lax.dot_general(g, ws_ref[...], (((1,), (0,)), ((), ())),
                             preferred_element_type=jnp.float32,
                             precision=jax.lax.Precision.HIGHEST)
    z_ref[...] = zm + eps_ref[...] * jnp.exp(zl)


def _dec_body(a_ref, b_ref, o_ref):
    m = jax.lax.dot_general(
        a_ref[...], b_ref[...], (((1,), (1,)), ((), ())),
        preferred_element_type=jnp.float32)
    for r in range(BM):
        o_ref[pl.ds(r * N, N)] = m[r, :]


BM = 128


@jax.jit
def kernel(x, edge_index, edge_weight, eps, W0, W_mean, W_std):
    src = edge_index[0].astype(jnp.int32)
    dst = edge_index[1].astype(jnp.int32)
    pad = E_PAD - E
    srcp = jnp.pad(src, (0, pad)).reshape(NW, CHUNKS, CHUNK)
    dstp = jnp.pad(dst, (0, pad)).reshape(NW, CHUNKS, CHUNK)
    wp = jnp.pad(edge_weight.astype(jnp.float32), (0, pad)).reshape(NW, EPW)
    zeros = jnp.zeros((ROWS_PER_SUB, H1), jnp.float32)

    h0 = pl.pallas_call(
        _mm_body,
        out_shape=jax.ShapeDtypeStruct((N, H1), jnp.float32),
    )(x, W0)

    p = _spmm(h0, srcp, dstp, wp, zeros)

    hidden1 = pl.pallas_call(
        _relu_combine_body,
        out_shape=jax.ShapeDtypeStruct((N, H1), jnp.float32),
    )(p)

    q = _spmm(hidden1, srcp, dstp, wp, zeros)

    z = pl.pallas_call(
        _z_body,
        out_shape=jax.ShapeDtypeStruct((N, H2), jnp.float32),
    )(q, W_mean, W_std, eps)

    rec = pl.pallas_call(
        _dec_body,
        grid=(pl.cdiv(N, BM),),
        in_specs=[
            pl.BlockSpec((BM, H2), lambda i: (i, 0)),
            pl.BlockSpec((N, H2), lambda i: (0, 0)),
        ],
        out_specs=pl.BlockSpec((BM * N,), lambda i: (i,)),
        out_shape=jax.ShapeDtypeStruct((N * N,), jnp.float32),
    )(z, z)

    return rec


# speedup vs baseline: 14.6362x; 1.0026x over previous
import functools

import jax
import jax.numpy as jnp
from jax import lax
from jax.experimental import pallas as pl
from jax.experimental.pallas import tpu as pltpu
from jax.experimental.pallas import tpu_sc as plsc

N = 10000
E = 320000
D = 128
H1 = 32
H2 = 16

NC = 2
NS = 16
NW = NC * NS
CHUNK = 128
CHUNKS = 80
EPW = CHUNKS * CHUNK
E_PAD = NW * EPW
ROWS_PER_SUB = 632
N_PAD = NS * ROWS_PER_SUB


def _spmm_body(combine, table_hbm, src_hbm, dst_hbm, w_hbm, zeros_hbm,
               out_hbm, src_v, dst_v, w_v, gbuf0, gbuf1, sbuf0, sbuf1,
               tbuf0, tbuf1, table_sh, accum_sh,
               sem_g0, sem_g1, sem_s0, sem_s1):
    c = lax.axis_index("c")
    s = lax.axis_index("s")
    wid = s * NC + c
    base = s * ROWS_PER_SUB

    pltpu.sync_copy(src_hbm.at[wid], src_v)
    pltpu.sync_copy(dst_hbm.at[wid], dst_v)
    pltpu.sync_copy(w_hbm.at[wid], w_v)

    if combine:
        pltpu.sync_copy(table_hbm.at[0, pl.ds(base, ROWS_PER_SUB)], tbuf0)
        pltpu.sync_copy(table_hbm.at[1, pl.ds(base, ROWS_PER_SUB)], tbuf1)

        @plsc.parallel_loop(0, ROWS_PER_SUB, 1, unroll=8)
        def _(r):
            t0 = tbuf0[r, pl.ds(0, 16)] + tbuf1[r, pl.ds(0, 16)]
            tbuf0[r, pl.ds(0, 16)] = jnp.maximum(t0, 0.0)
            t1 = tbuf0[r, pl.ds(16, 16)] + tbuf1[r, pl.ds(16, 16)]
            tbuf0[r, pl.ds(16, 16)] = jnp.maximum(t1, 0.0)

        pltpu.sync_copy(tbuf0, table_sh.at[pl.ds(base, ROWS_PER_SUB)])
    else:
        pltpu.sync_copy(table_hbm.at[pl.ds(base, ROWS_PER_SUB)],
                        table_sh.at[pl.ds(base, ROWS_PER_SUB)])

    pltpu.sync_copy(zeros_hbm,
                    accum_sh.at[pl.ds(s * ROWS_PER_SUB, ROWS_PER_SUB)])
    plsc.subcore_barrier()

    def scale(j, gbuf, sbuf):
        jj = jnp.full((16,), j * CHUNK, jnp.int32)

        @plsc.parallel_loop(0, CHUNK, 1, unroll=8)
        def _(e):
            wv = plsc.load_gather(w_v, [jj + e])
            sbuf[e, pl.ds(0, 16)] = gbuf[e, pl.ds(0, 16)] * wv
            sbuf[e, pl.ds(16, 16)] = gbuf[e, pl.ds(16, 16)] * wv

    def gather_start(j, gbuf, sem):
        pltpu.async_copy(table_sh.at[src_v.at[j]], gbuf, sem)

    def gather_wait(j, gbuf, sem):
        pltpu.make_async_copy(table_sh.at[src_v.at[j]], gbuf, sem).wait()

    def scatter_start(j, sbuf, sem):
        pltpu.async_copy(sbuf, accum_sh.at[dst_v.at[j]], sem, add=True)

    def scatter_wait(j, sbuf, sem):
        pltpu.make_async_copy(sbuf, accum_sh.at[dst_v.at[j]], sem).wait()

    HALF = CHUNKS // 2

    gather_start(0, gbuf0, sem_g0)
    gather_start(1, gbuf1, sem_g1)

    def pair_body(h, carry):
        j0 = h * 2
        j1 = j0 + 1

        gather_wait(j0, gbuf0, sem_g0)

        @pl.when(h > 0)
        def _():
            scatter_wait(j0 - 2, sbuf0, sem_s0)

        scale(j0, gbuf0, sbuf0)

        @pl.when(h < HALF - 1)
        def _():
            gather_start(j0 + 2, gbuf0, sem_g0)

        scatter_start(j0, sbuf0, sem_s0)

        gather_wait(j1, gbuf1, sem_g1)

        @pl.when(h > 0)
        def _():
            scatter_wait(j1 - 2, sbuf1, sem_s1)

        scale(j1, gbuf1, sbuf1)

        @pl.when(h < HALF - 1)
        def _():
            gather_start(j1 + 2, gbuf1, sem_g1)

        scatter_start(j1, sbuf1, sem_s1)
        return carry

    lax.fori_loop(0, HALF, pair_body, 0)
    scatter_wait(CHUNKS - 2, sbuf0, sem_s0)
    scatter_wait(CHUNKS - 1, sbuf1, sem_s1)
    plsc.subcore_barrier()

    pltpu.sync_copy(accum_sh.at[pl.ds(s * ROWS_PER_SUB, ROWS_PER_SUB)],
                    out_hbm.at[c, pl.ds(s * ROWS_PER_SUB, ROWS_PER_SUB)])


def _spmm(table, srcp, dstp, wp, zeros, combine):
    mesh = plsc.VectorSubcoreMesh(core_axis_name="c", subcore_axis_name="s")
    f = pl.kernel(
        functools.partial(_spmm_body, combine),
        out_type=jax.ShapeDtypeStruct((NC, N_PAD, H1), jnp.float32),
        mesh=mesh,
        scratch_types=[
            pltpu.VMEM((CHUNKS, CHUNK), jnp.int32),
            pltpu.VMEM((CHUNKS, CHUNK), jnp.int32),
            pltpu.VMEM((EPW,), jnp.float32),
            pltpu.VMEM((CHUNK, H1), jnp.float32),
            pltpu.VMEM((CHUNK, H1), jnp.float32),
            pltpu.VMEM((CHUNK, H1), jnp.float32),
            pltpu.VMEM((CHUNK, H1), jnp.float32),
            pltpu.VMEM((ROWS_PER_SUB, H1), jnp.float32),
            pltpu.VMEM((ROWS_PER_SUB, H1), jnp.float32),
            pltpu.VMEM_SHARED((N_PAD, H1), jnp.float32),
            pltpu.VMEM_SHARED((N_PAD, H1), jnp.float32),
            pltpu.SemaphoreType.DMA,
            pltpu.SemaphoreType.DMA,
            pltpu.SemaphoreType.DMA,
            pltpu.SemaphoreType.DMA,
        ],
        compiler_params=pltpu.CompilerParams(
            needs_layout_passes=False, use_tc_tiling_on_sc=False),
    )
    return f(table, srcp, dstp, wp, zeros)


def _mm_body(x_ref, w_ref, o_ref):
    o_ref[pl.ds(0, N), :] = jax.lax.dot_general(
        x_ref[...], w_ref[...], (((1,), (0,)), ((), ())),
        preferred_element_type=jnp.float32,
        precision=jax.lax.Precision.HIGHEST)
    o_ref[pl.ds(N, N_PAD - N), :] = jnp.zeros((N_PAD - N, H1), jnp.float32)


def _z_body(q_ref, wm_ref, ws_ref, eps_ref, z_ref):
    g = q_ref[0, :N, :] + q_ref[1, :N, :]
    zm = jax.lax.dot_general(g, wm_ref[...], (((1,), (0,)), ((), ())),
                             preferred_element_type=jnp.float32,
                             precision=jax.lax.Precision.HIGHEST)
    zl = jax.lax.dot_general(g, ws_ref[...], (((1,), (0,)), ((), ())),
                             preferred_element_type=jnp.float32,
                             precision=jax.lax.Precision.HIGHEST)
    z_ref[...] = zm + eps_ref[...] * jnp.exp(zl)


def _dec_body(a_ref, b_ref, o_ref):
    m = jax.lax.dot_general(
        a_ref[...], b_ref[...], (((1,), (1,)), ((), ())),
        preferred_element_type=jnp.float32)
    for r in range(BM):
        o_ref[pl.ds(r * N, N)] = m[r, :]


BM = 128


@jax.jit
def kernel(x, edge_index, edge_weight, eps, W0, W_mean, W_std):
    src = edge_index[0].astype(jnp.int32)
    dst = edge_index[1].astype(jnp.int32)
    pad = E_PAD - E
    srcp = jnp.pad(src, (0, pad)).reshape(NW, CHUNKS, CHUNK)
    dstp = jnp.pad(dst, (0, pad)).reshape(NW, CHUNKS, CHUNK)
    wp = jnp.pad(edge_weight.astype(jnp.float32), (0, pad)).reshape(NW, EPW)
    zeros = jnp.zeros((ROWS_PER_SUB, H1), jnp.float32)

    h0 = pl.pallas_call(
        _mm_body,
        out_shape=jax.ShapeDtypeStruct((N_PAD, H1), jnp.float32),
    )(x, W0)

    p = _spmm(h0, srcp, dstp, wp, zeros, combine=False)
    q = _spmm(p, srcp, dstp, wp, zeros, combine=True)

    z = pl.pallas_call(
        _z_body,
        out_shape=jax.ShapeDtypeStruct((N, H2), jnp.float32),
    )(q, W_mean, W_std, eps)

    rec = pl.pallas_call(
        _dec_body,
        grid=(pl.cdiv(N, BM),),
        in_specs=[
            pl.BlockSpec((BM, H2), lambda i: (i, 0)),
            pl.BlockSpec((N, H2), lambda i: (0, 0)),
        ],
        out_specs=pl.BlockSpec((BM * N,), lambda i: (i,)),
        out_shape=jax.ShapeDtypeStruct((N * N,), jnp.float32),
    )(z, z)

    return rec


# speedup vs baseline: 15.6499x; 1.0693x over previous
import functools

import jax
import jax.numpy as jnp
from jax import lax
from jax.experimental import pallas as pl
from jax.experimental.pallas import tpu as pltpu
from jax.experimental.pallas import tpu_sc as plsc

N = 10000
E = 320000
D = 128
H1 = 32
H2 = 16

NC = 2
NS = 16
NW = NC * NS
CHUNK = 128
CHUNKS = 80
EPW = CHUNKS * CHUNK
E_PAD = NW * EPW
ROWS_PER_SUB = 632
N_PAD = NS * ROWS_PER_SUB


def _spmm_body(combine, table_hbm, src_hbm, dst_hbm, w_hbm, zeros_hbm,
               out_hbm, src_v, dst_v, w_v, gbuf0, gbuf1, sbuf0, sbuf1,
               tbuf0, tbuf1, table_sh, accum_sh,
               sem_g0, sem_g1, sem_s0, sem_s1):
    c = lax.axis_index("c")
    s = lax.axis_index("s")
    wid = s * NC + c
    base = s * ROWS_PER_SUB

    pltpu.sync_copy(src_hbm.at[wid], src_v)
    pltpu.sync_copy(dst_hbm.at[wid], dst_v)
    pltpu.sync_copy(w_hbm.at[wid], w_v)

    if combine:
        pltpu.sync_copy(table_hbm.at[0, pl.ds(base, ROWS_PER_SUB)], tbuf0)
        pltpu.sync_copy(table_hbm.at[1, pl.ds(base, ROWS_PER_SUB)], tbuf1)

        @plsc.parallel_loop(0, ROWS_PER_SUB, 1, unroll=8)
        def _(r):
            t0 = tbuf0[r, pl.ds(0, 16)] + tbuf1[r, pl.ds(0, 16)]
            tbuf0[r, pl.ds(0, 16)] = jnp.maximum(t0, 0.0)
            t1 = tbuf0[r, pl.ds(16, 16)] + tbuf1[r, pl.ds(16, 16)]
            tbuf0[r, pl.ds(16, 16)] = jnp.maximum(t1, 0.0)

        pltpu.sync_copy(tbuf0, table_sh.at[pl.ds(base, ROWS_PER_SUB)])
    else:
        pltpu.sync_copy(table_hbm.at[pl.ds(base, ROWS_PER_SUB)],
                        table_sh.at[pl.ds(base, ROWS_PER_SUB)])

    pltpu.sync_copy(zeros_hbm,
                    accum_sh.at[pl.ds(s * ROWS_PER_SUB, ROWS_PER_SUB)])
    plsc.subcore_barrier()

    def scale(j, gbuf, sbuf):
        jj = jnp.full((16,), j * CHUNK, jnp.int32)

        @plsc.parallel_loop(0, CHUNK, 1, unroll=8)
        def _(e):
            wv = plsc.load_gather(w_v, [jj + e])
            sbuf[e, pl.ds(0, 16)] = gbuf[e, pl.ds(0, 16)] * wv
            sbuf[e, pl.ds(16, 16)] = gbuf[e, pl.ds(16, 16)] * wv

    def gather_start(j, gbuf, sem):
        pltpu.async_copy(table_sh.at[src_v.at[j]], gbuf, sem)

    def gather_wait(j, gbuf, sem):
        pltpu.make_async_copy(table_sh.at[src_v.at[j]], gbuf, sem).wait()

    def scatter_start(j, sbuf, sem):
        pltpu.async_copy(sbuf, accum_sh.at[dst_v.at[j]], sem, add=True)

    def scatter_wait(j, sbuf, sem):
        pltpu.make_async_copy(sbuf, accum_sh.at[dst_v.at[j]], sem).wait()

    HALF = CHUNKS // 2

    gather_start(0, gbuf0, sem_g0)
    gather_start(1, gbuf1, sem_g1)

    def pair_body(h, carry):
        j0 = h * 2
        j1 = j0 + 1

        gather_wait(j0, gbuf0, sem_g0)

        @pl.when(h > 0)
        def _():
            scatter_wait(j0 - 2, sbuf0, sem_s0)

        scale(j0, gbuf0, sbuf0)

        @pl.when(h < HALF - 1)
        def _():
            gather_start(j0 + 2, gbuf0, sem_g0)

        scatter_start(j0, sbuf0, sem_s0)

        gather_wait(j1, gbuf1, sem_g1)

        @pl.when(h > 0)
        def _():
            scatter_wait(j1 - 2, sbuf1, sem_s1)

        scale(j1, gbuf1, sbuf1)

        @pl.when(h < HALF - 1)
        def _():
            gather_start(j1 + 2, gbuf1, sem_g1)

        scatter_start(j1, sbuf1, sem_s1)
        return carry

    lax.fori_loop(0, HALF, pair_body, 0)
    scatter_wait(CHUNKS - 2, sbuf0, sem_s0)
    scatter_wait(CHUNKS - 1, sbuf1, sem_s1)
    plsc.subcore_barrier()

    pltpu.sync_copy(accum_sh.at[pl.ds(s * ROWS_PER_SUB, ROWS_PER_SUB)],
                    out_hbm.at[c, pl.ds(s * ROWS_PER_SUB, ROWS_PER_SUB)])


def _spmm(table, srcp, dstp, wp, zeros, combine):
    mesh = plsc.VectorSubcoreMesh(core_axis_name="c", subcore_axis_name="s")
    f = pl.kernel(
        functools.partial(_spmm_body, combine),
        out_type=jax.ShapeDtypeStruct((NC, N_PAD, H1), jnp.float32),
        mesh=mesh,
        scratch_types=[
            pltpu.VMEM((CHUNKS, CHUNK), jnp.int32),
            pltpu.VMEM((CHUNKS, CHUNK), jnp.int32),
            pltpu.VMEM((EPW,), jnp.float32),
            pltpu.VMEM((CHUNK, H1), jnp.float32),
            pltpu.VMEM((CHUNK, H1), jnp.float32),
            pltpu.VMEM((CHUNK, H1), jnp.float32),
            pltpu.VMEM((CHUNK, H1), jnp.float32),
            pltpu.VMEM((ROWS_PER_SUB, H1), jnp.float32),
            pltpu.VMEM((ROWS_PER_SUB, H1), jnp.float32),
            pltpu.VMEM_SHARED((N_PAD, H1), jnp.float32),
            pltpu.VMEM_SHARED((N_PAD, H1), jnp.float32),
            pltpu.SemaphoreType.DMA,
            pltpu.SemaphoreType.DMA,
            pltpu.SemaphoreType.DMA,
            pltpu.SemaphoreType.DMA,
        ],
        compiler_params=pltpu.CompilerParams(
            needs_layout_passes=False, use_tc_tiling_on_sc=False),
    )
    return f(table, srcp, dstp, wp, zeros)


def _mm_body(x_ref, w_ref, o_ref):
    o_ref[pl.ds(0, N), :] = jax.lax.dot_general(
        x_ref[...], w_ref[...], (((1,), (0,)), ((), ())),
        preferred_element_type=jnp.float32)
    o_ref[pl.ds(N, N_PAD - N), :] = jnp.zeros((N_PAD - N, H1), jnp.float32)


def _z_body(q_ref, wm_ref, ws_ref, eps_ref, z_ref):
    g = q_ref[0, :N, :] + q_ref[1, :N, :]
    zm = jax.lax.dot_general(g, wm_ref[...], (((1,), (0,)), ((), ())),
                             preferred_element_type=jnp.float32)
    zl = jax.lax.dot_general(g, ws_ref[...], (((1,), (0,)), ((), ())),
                             preferred_element_type=jnp.float32)
    z_ref[...] = zm + eps_ref[...] * jnp.exp(zl)


def _dec_body(a_ref, b_ref, o_ref):
    m = jax.lax.dot_general(
        a_ref[...], b_ref[...], (((1,), (1,)), ((), ())),
        preferred_element_type=jnp.float32)
    for r in range(BM):
        o_ref[pl.ds(r * N, N)] = m[r, :]


BM = 256


@jax.jit
def kernel(x, edge_index, edge_weight, eps, W0, W_mean, W_std):
    src = edge_index[0].astype(jnp.int32)
    dst = edge_index[1].astype(jnp.int32)
    pad = E_PAD - E
    srcp = jnp.pad(src, (0, pad)).reshape(NW, CHUNKS, CHUNK)
    dstp = jnp.pad(dst, (0, pad)).reshape(NW, CHUNKS, CHUNK)
    wp = jnp.pad(edge_weight.astype(jnp.float32), (0, pad)).reshape(NW, EPW)
    zeros = jnp.zeros((ROWS_PER_SUB, H1), jnp.float32)

    h0 = pl.pallas_call(
        _mm_body,
        out_shape=jax.ShapeDtypeStruct((N_PAD, H1), jnp.float32),
    )(x, W0)

    p = _spmm(h0, srcp, dstp, wp, zeros, combine=False)
    q = _spmm(p, srcp, dstp, wp, zeros, combine=True)

    z = pl.pallas_call(
        _z_body,
        out_shape=jax.ShapeDtypeStruct((N, H2), jnp.float32),
    )(q, W_mean, W_std, eps)

    rec = pl.pallas_call(
        _dec_body,
        grid=(pl.cdiv(N, BM),),
        in_specs=[
            pl.BlockSpec((BM, H2), lambda i: (i, 0)),
            pl.BlockSpec((N, H2), lambda i: (0, 0)),
        ],
        out_specs=pl.BlockSpec((BM * N,), lambda i: (i,)),
        out_shape=jax.ShapeDtypeStruct((N * N,), jnp.float32),
    )(z, z)

    return rec


# speedup vs baseline: 15.9450x; 1.0189x over previous
import functools

import jax
import jax.numpy as jnp
from jax import lax
from jax.experimental import pallas as pl
from jax.experimental.pallas import tpu as pltpu
from jax.experimental.pallas import tpu_sc as plsc

N = 10000
E = 320000
D = 128
H1 = 32
H2 = 16

NC = 2
NS = 16
NW = NC * NS
CHUNK = 125
CHUNKS = 80
EPW = CHUNKS * CHUNK
ROWS_PER_SUB = 632
N_PAD = NS * ROWS_PER_SUB


def _spmm_body(combine, table_hbm, src_hbm, dst_hbm, w_hbm, zeros_hbm,
               out_hbm, src_v, dst_v, w_v, gbuf0, gbuf1, sbuf0, sbuf1,
               tbuf0, tbuf1, table_sh, accum_sh,
               sem_g0, sem_g1, sem_s0, sem_s1):
    c = lax.axis_index("c")
    s = lax.axis_index("s")
    wid = s * NC + c
    base = s * ROWS_PER_SUB

    pltpu.sync_copy(src_hbm.at[wid], src_v)
    pltpu.sync_copy(dst_hbm.at[wid], dst_v)
    pltpu.sync_copy(w_hbm.at[wid], w_v)

    if combine:
        pltpu.sync_copy(table_hbm.at[0, pl.ds(base, ROWS_PER_SUB)], tbuf0)
        pltpu.sync_copy(table_hbm.at[1, pl.ds(base, ROWS_PER_SUB)], tbuf1)

        @plsc.parallel_loop(0, ROWS_PER_SUB, 1, unroll=8)
        def _(r):
            t0 = tbuf0[r, pl.ds(0, 16)] + tbuf1[r, pl.ds(0, 16)]
            tbuf0[r, pl.ds(0, 16)] = jnp.maximum(t0, 0.0)
            t1 = tbuf0[r, pl.ds(16, 16)] + tbuf1[r, pl.ds(16, 16)]
            tbuf0[r, pl.ds(16, 16)] = jnp.maximum(t1, 0.0)

        pltpu.sync_copy(tbuf0, table_sh.at[pl.ds(base, ROWS_PER_SUB)])
    else:
        pltpu.sync_copy(table_hbm.at[pl.ds(base, ROWS_PER_SUB)],
                        table_sh.at[pl.ds(base, ROWS_PER_SUB)])

    pltpu.sync_copy(zeros_hbm,
                    accum_sh.at[pl.ds(s * ROWS_PER_SUB, ROWS_PER_SUB)])
    plsc.subcore_barrier()

    def scale(j, gbuf, sbuf):
        jj = jnp.full((16,), j * CHUNK, jnp.int32)

        @plsc.parallel_loop(0, CHUNK, 1, unroll=5)
        def _(e):
            wv = plsc.load_gather(w_v, [jj + e])
            sbuf[e, pl.ds(0, 16)] = gbuf[e, pl.ds(0, 16)] * wv
            sbuf[e, pl.ds(16, 16)] = gbuf[e, pl.ds(16, 16)] * wv

    def gather_start(j, gbuf, sem):
        pltpu.async_copy(table_sh.at[src_v.at[j]], gbuf, sem)

    def gather_wait(j, gbuf, sem):
        pltpu.make_async_copy(table_sh.at[src_v.at[j]], gbuf, sem).wait()

    def scatter_start(j, sbuf, sem):
        pltpu.async_copy(sbuf, accum_sh.at[dst_v.at[j]], sem, add=True)

    def scatter_wait(j, sbuf, sem):
        pltpu.make_async_copy(sbuf, accum_sh.at[dst_v.at[j]], sem).wait()

    HALF = CHUNKS // 2

    gather_start(0, gbuf0, sem_g0)
    gather_start(1, gbuf1, sem_g1)

    def pair_body(h, carry):
        j0 = h * 2
        j1 = j0 + 1

        gather_wait(j0, gbuf0, sem_g0)

        @pl.when(h > 0)
        def _():
            scatter_wait(j0 - 2, sbuf0, sem_s0)

        scale(j0, gbuf0, sbuf0)

        @pl.when(h < HALF - 1)
        def _():
            gather_start(j0 + 2, gbuf0, sem_g0)

        scatter_start(j0, sbuf0, sem_s0)

        gather_wait(j1, gbuf1, sem_g1)

        @pl.when(h > 0)
        def _():
            scatter_wait(j1 - 2, sbuf1, sem_s1)

        scale(j1, gbuf1, sbuf1)

        @pl.when(h < HALF - 1)
        def _():
            gather_start(j1 + 2, gbuf1, sem_g1)

        scatter_start(j1, sbuf1, sem_s1)
        return carry

    lax.fori_loop(0, HALF, pair_body, 0)
    scatter_wait(CHUNKS - 2, sbuf0, sem_s0)
    scatter_wait(CHUNKS - 1, sbuf1, sem_s1)
    plsc.subcore_barrier()

    pltpu.sync_copy(accum_sh.at[pl.ds(s * ROWS_PER_SUB, ROWS_PER_SUB)],
                    out_hbm.at[c, pl.ds(s * ROWS_PER_SUB, ROWS_PER_SUB)])


def _spmm(table, srcp, dstp, wp, zeros, combine):
    mesh = plsc.VectorSubcoreMesh(core_axis_name="c", subcore_axis_name="s")
    f = pl.kernel(
        functools.partial(_spmm_body, combine),
        out_type=jax.ShapeDtypeStruct((NC, N_PAD, H1), jnp.float32),
        mesh=mesh,
        scratch_types=[
            pltpu.VMEM((CHUNKS, CHUNK), jnp.int32),
            pltpu.VMEM((CHUNKS, CHUNK), jnp.int32),
            pltpu.VMEM((EPW,), jnp.float32),
            pltpu.VMEM((CHUNK, H1), jnp.float32),
            pltpu.VMEM((CHUNK, H1), jnp.float32),
            pltpu.VMEM((CHUNK, H1), jnp.float32),
            pltpu.VMEM((CHUNK, H1), jnp.float32),
            pltpu.VMEM((ROWS_PER_SUB, H1), jnp.float32),
            pltpu.VMEM((ROWS_PER_SUB, H1), jnp.float32),
            pltpu.VMEM_SHARED((N_PAD, H1), jnp.float32),
            pltpu.VMEM_SHARED((N_PAD, H1), jnp.float32),
            pltpu.SemaphoreType.DMA,
            pltpu.SemaphoreType.DMA,
            pltpu.SemaphoreType.DMA,
            pltpu.SemaphoreType.DMA,
        ],
        compiler_params=pltpu.CompilerParams(
            needs_layout_passes=False, use_tc_tiling_on_sc=False),
    )
    return f(table, srcp, dstp, wp, zeros)


def _mm_body(x_ref, w_ref, o_ref):
    o_ref[pl.ds(0, N), :] = jax.lax.dot_general(
        x_ref[...], w_ref[...], (((1,), (0,)), ((), ())),
        preferred_element_type=jnp.float32)
    o_ref[pl.ds(N, N_PAD - N), :] = jnp.zeros((N_PAD - N, H1), jnp.float32)


def _z_body(q_ref, wm_ref, ws_ref, eps_ref, z_ref):
    g = q_ref[0, :N, :] + q_ref[1, :N, :]
    zm = jax.lax.dot_general(g, wm_ref[...], (((1,), (0,)), ((), ())),
                             preferred_element_type=jnp.float32)
    zl = jax.lax.dot_general(g, ws_ref[...], (((1,), (0,)), ((), ())),
                             preferred_element_type=jnp.float32)
    z_ref[...] = zm + eps_ref[...] * jnp.exp(zl)


def _dec_body(a_ref, b_ref, o_ref):
    m = jax.lax.dot_general(
        a_ref[...], b_ref[...], (((1,), (1,)), ((), ())),
        preferred_element_type=jnp.float32)
    for r in range(BM):
        o_ref[pl.ds(r * N, N)] = m[r, :]


BM = 256


@jax.jit
def kernel(x, edge_index, edge_weight, eps, W0, W_mean, W_std):
    srcp = edge_index[0].astype(jnp.int32).reshape(NW, CHUNKS, CHUNK)
    dstp = edge_index[1].astype(jnp.int32).reshape(NW, CHUNKS, CHUNK)
    wp = edge_weight.astype(jnp.float32).reshape(NW, EPW)
    zeros = jnp.zeros((ROWS_PER_SUB, H1), jnp.float32)

    h0 = pl.pallas_call(
        _mm_body,
        out_shape=jax.ShapeDtypeStruct((N_PAD, H1), jnp.float32),
    )(x, W0)

    p = _spmm(h0, srcp, dstp, wp, zeros, combine=False)
    q = _spmm(p, srcp, dstp, wp, zeros, combine=True)

    z = pl.pallas_call(
        _z_body,
        out_shape=jax.ShapeDtypeStruct((N, H2), jnp.float32),
    )(q, W_mean, W_std, eps)

    rec = pl.pallas_call(
        _dec_body,
        grid=(pl.cdiv(N, BM),),
        in_specs=[
            pl.BlockSpec((BM, H2), lambda i: (i, 0)),
            pl.BlockSpec((N, H2), lambda i: (0, 0)),
        ],
        out_specs=pl.BlockSpec((BM * N,), lambda i: (i,)),
        out_shape=jax.ShapeDtypeStruct((N * N,), jnp.float32),
    )(z, z)

    return rec
